# baseline (speedup 1.0000x reference)
import functools
import math

import jax
import jax.numpy as jnp
from jax import lax
from jax.experimental import pallas as pl
from jax.experimental.pallas import tpu as pltpu
from jax.experimental.pallas import tpu_sc as plsc

_EMBED_DIM = 1024
_PADDING_IDX = 1
_FRONT_PAD = 8 - (_PADDING_IDX + 1)
_ZERO_ROW = _PADDING_IDX + _FRONT_PAD


def _build_table(num_embeddings: int, embed_dim: int, padding_idx: int):
    half = embed_dim // 2
    scale = math.log(10000.0) / (half - 1)
    inv = jnp.exp(jnp.arange(half, dtype=jnp.float32) * -scale)
    pos = jnp.arange(num_embeddings, dtype=jnp.float32)
    ang = pos[:, None] * inv[None, :]
    emb = jnp.concatenate([jnp.sin(ang), jnp.cos(ang)], axis=1)
    emb = emb.at[padding_idx, :].set(0.0)
    return emb


def kernel(x):
    bsz, seq_len = x.shape
    n_rows = bsz * seq_len
    table = _build_table(_PADDING_IDX + 1 + seq_len, _EMBED_DIM, _PADDING_IDX)
    table = jnp.concatenate(
        [jnp.zeros((_FRONT_PAD, _EMBED_DIM), jnp.float32), table], axis=0
    )
    xf = x.reshape(n_rows)

    info = plsc.get_sparse_core_info()
    nc, ns, lanes = info.num_cores, info.num_subcores, info.num_lanes
    nw = nc * ns
    t_per_w = seq_len // nw
    chunk = 32
    n_chunks = t_per_w // chunk
    nbuf = 3

    mesh = plsc.VectorSubcoreMesh(core_axis_name="c", subcore_axis_name="s")

    @functools.partial(
        pl.kernel,
        mesh=mesh,
        out_type=jax.ShapeDtypeStruct((n_rows, _EMBED_DIM), jnp.float32),
        scratch_types=[
            pltpu.VMEM((bsz, t_per_w), jnp.int32),
            pltpu.VMEM((lanes,), jnp.int32),
            pltpu.VMEM((lanes,), jnp.int32),
            pltpu.VMEM_SHARED((ns, nbuf, chunk, _EMBED_DIM), jnp.float32),
            pltpu.VMEM((lanes, _EMBED_DIM), jnp.float32),
            pltpu.SemaphoreType.DMA,
            pltpu.SemaphoreType.DMA,
            pltpu.SemaphoreType.DMA,
        ],
    )
    def sc_kernel(table_hbm, x_hbm, out_hbm, xv, zidx, pidx, shbuf, zbuf, gsem, ssem, psem):
        wid = lax.axis_index("s") * nc + lax.axis_index("c")
        sid = lax.axis_index("s")
        t0w = wid * t_per_w
        perms = [lax.iota(jnp.int32, lanes) ^ s for s in (1, 2, 4, 8)]

        def gather(ci):
            off = t0w + ci * chunk + _PADDING_IDX + 1 + _FRONT_PAD
            return pltpu.make_async_copy(
                table_hbm.at[pl.ds(off, chunk)], shbuf.at[sid, ci % nbuf], gsem
            )

        def scatter(b, ci):
            return pltpu.make_async_copy(
                shbuf.at[sid, ci % nbuf],
                out_hbm.at[pl.ds(b * seq_len + t0w + ci * chunk, chunk)],
                ssem,
            )

        for ci in range(min(2, n_chunks)):
            gather(ci).start()
        for ci in range(n_chunks):
            gather(ci).wait()
            for b in range(bsz):
                scatter(b, ci).start()
            if ci + 2 < n_chunks:
                if ci >= 1:
                    for b in range(bsz):
                        scatter(b, ci - 1).wait()
                gather(ci + 2).start()
        for ci in range(max(0, n_chunks - 3), n_chunks):
            for b in range(bsz):
                scatter(b, ci).wait()

        for b in range(bsz):
            pltpu.sync_copy(x_hbm.at[pl.ds(b * seq_len + t0w, t_per_w)], xv.at[b])
        zidx[...] = jnp.full((lanes,), _ZERO_ROW, jnp.int32)
        pltpu.async_copy(table_hbm.at[zidx], zbuf, psem).wait()

        for ci in range(n_chunks):
            tb = ci * chunk
            for b in range(bsz):
                acc = None
                for i in range(chunk // lanes):
                    toks = xv[b, pl.ds(tb + i * lanes, lanes)]
                    m = jnp.where(toks == _PADDING_IDX, 1, 0)
                    acc = m if acc is None else acc | m
                for perm in perms:
                    acc = jnp.maximum(
                        acc, acc.at[perm].get(mode="promise_in_bounds")
                    )
                has_pad = acc[0] > 0

                def patch(tb=tb, b=b):
                    big = jnp.int32(1 << 30)
                    first = None
                    for i in range(chunk // lanes):
                        toks = xv[b, pl.ds(tb + i * lanes, lanes)]
                        rows = lax.iota(jnp.int32, lanes) + (
                            b * seq_len + t0w + tb + i * lanes
                        )
                        cand = jnp.where(toks == _PADDING_IDX, rows, big)
                        first = cand if first is None else jnp.minimum(first, cand)
                    for perm in perms:
                        first = jnp.minimum(
                            first, first.at[perm].get(mode="promise_in_bounds")
                        )
                    for i in range(chunk // lanes):
                        toks = xv[b, pl.ds(tb + i * lanes, lanes)]
                        rows = lax.iota(jnp.int32, lanes) + (
                            b * seq_len + t0w + tb + i * lanes
                        )
                        pidx[...] = jnp.where(toks == _PADDING_IDX, rows, first)
                        pltpu.async_copy(zbuf, out_hbm.at[pidx], psem).wait()

                lax.cond(has_pad, patch, lambda: None)

    out = sc_kernel(table, xf)
    return out.reshape(bsz, seq_len, _EMBED_DIM)


# speedup vs baseline: 1.1066x; 1.1066x over previous
import functools
import math

import jax
import jax.numpy as jnp
from jax import lax
from jax.experimental import pallas as pl
from jax.experimental.pallas import tpu as pltpu
from jax.experimental.pallas import tpu_sc as plsc

_EMBED_DIM = 1024
_PADDING_IDX = 1


def _build_table(num_embeddings: int, embed_dim: int, padding_idx: int):
    half = embed_dim // 2
    scale = math.log(10000.0) / (half - 1)
    inv = jnp.exp(jnp.arange(half, dtype=jnp.float32) * -scale)
    pos = jnp.arange(num_embeddings, dtype=jnp.float32)
    ang = pos[:, None] * inv[None, :]
    emb = jnp.concatenate([jnp.sin(ang), jnp.cos(ang)], axis=1)
    emb = emb.at[padding_idx, :].set(0.0)
    return emb


def kernel(x):
    bsz, seq_len = x.shape
    n_rows = bsz * seq_len
    table = _build_table(_PADDING_IDX + 1 + seq_len, _EMBED_DIM, _PADDING_IDX)
    xf = x.reshape(n_rows)

    info = plsc.get_sparse_core_info()
    nc, ns, lanes = info.num_cores, info.num_subcores, info.num_lanes
    nw = nc * ns
    rows_per_w = n_rows // nw
    chunk = 32
    n_chunks = rows_per_w // chunk
    nbuf = 3

    mesh = plsc.VectorSubcoreMesh(core_axis_name="c", subcore_axis_name="s")

    @functools.partial(
        pl.kernel,
        mesh=mesh,
        out_type=jax.ShapeDtypeStruct((n_rows, _EMBED_DIM), jnp.float32),
        scratch_types=[
            pltpu.VMEM((rows_per_w,), jnp.int32),
            pltpu.VMEM((n_chunks, chunk), jnp.int32),
            pltpu.VMEM((nbuf, chunk, _EMBED_DIM), jnp.float32),
            pltpu.SemaphoreType.DMA,
            pltpu.SemaphoreType.DMA,
        ],
    )
    def sc_kernel(table_hbm, x_hbm, out_hbm, xv, idxv, buf, gsem, ssem):
        wid = lax.axis_index("s") * nc + lax.axis_index("c")
        wbase = wid * rows_per_w
        tbase = lax.rem(wbase, seq_len)

        pltpu.sync_copy(x_hbm.at[pl.ds(wbase, rows_per_w)], xv)
        for i in range(rows_per_w // lanes):
            toks = xv[pl.ds(i * lanes, lanes)]
            seq_pos = lax.iota(jnp.int32, lanes) + (
                tbase + i * lanes + _PADDING_IDX + 1
            )
            p = jnp.where(toks != _PADDING_IDX, seq_pos, _PADDING_IDX)
            ci, j = divmod(i * lanes, chunk)
            idxv[ci, pl.ds(j, lanes)] = p

        def gather(ci):
            return pltpu.make_async_copy(
                table_hbm.at[idxv.at[ci]], buf.at[ci % nbuf], gsem
            )

        def scatter(ci):
            return pltpu.make_async_copy(
                buf.at[ci % nbuf],
                out_hbm.at[pl.ds(wbase + ci * chunk, chunk)],
                ssem,
            )

        for ci in range(min(2, n_chunks)):
            gather(ci).start()
        for ci in range(n_chunks):
            gather(ci).wait()
            scatter(ci).start()
            if ci + 2 < n_chunks:
                if ci >= 1:
                    scatter(ci - 1).wait()
                gather(ci + 2).start()
        for ci in range(max(0, n_chunks - 3), n_chunks):
            scatter(ci).wait()

    out = sc_kernel(table, xf)
    return out.reshape(bsz, seq_len, _EMBED_DIM)
